# baseline (speedup 1.0000x reference)
import jax
import jax.numpy as jnp
from jax import lax
from jax.experimental import pallas as pl
from jax.experimental.pallas import tpu as pltpu
from jax.experimental.pallas import tpu_sc as plsc

B = 8
H = 16
N = 512
V = 513
L = 16
G = 4
ROWS_PER_TILE = (B * N) // 32
NCHUNK = ROWS_PER_TILE // G


def _sc_body(ab_hbm, sp_hbm, tabt_hbm, out_hbm, tab_v, ab_v, idx_v, out_v,
             tab_sem, ab_sems, idx_sems, out_sems):
    nc = 2
    wid = lax.axis_index("s") * nc + lax.axis_index("c")
    b = wid // (N // ROWS_PER_TILE)
    i0 = (wid % (N // ROWS_PER_TILE)) * ROWS_PER_TILE

    pltpu.async_copy(tabt_hbm, tab_v, tab_sem).wait()

    def in_descs(c, s):
        i = i0 + c * G
        return (
            pltpu.make_async_copy(ab_hbm.at[b, pl.ds(i, G), :], ab_v.at[s], ab_sems[s]),
            pltpu.make_async_copy(sp_hbm.at[b, pl.ds(i, G), :], idx_v.at[s], idx_sems[s]),
        )

    def out_desc(c, s):
        i = i0 + c * G
        return pltpu.make_async_copy(
            out_v.at[s], out_hbm.at[b, :, pl.ds(i, G), :], out_sems[s])

    def start_in(c, s):
        for d in in_descs(c, s):
            d.start()

    def wait_in(c, s):
        for d in in_descs(c, s):
            d.wait()

    def compute(s):
        for r in range(G):
            @plsc.parallel_loop(0, N // L, unroll=8)
            def _vec_body(v):
                sl = pl.ds(v * L, L)
                idx = idx_v[s, r, sl]
                ab2 = ab_v[s, r, sl] * 2.0
                for h in range(H):
                    g = plsc.load_gather(tab_v, [idx + (h * V)])
                    out_v[s, h, r, sl] = ab2 + g

    start_in(0, 0)
    start_in(1, 1)

    def steady(c2, carry):
        for s in (0, 1):
            c = 2 * c2 + s

            @pl.when(c >= 2)
            def _wait_out():
                out_desc(c - 2, s).wait()

            wait_in(c, s)
            compute(s)
            out_desc(c, s).start()

            @pl.when(c < NCHUNK - 2)
            def _prefetch():
                start_in(c + 2, s)
        return carry

    lax.fori_loop(0, NCHUNK // 2, steady, 0)

    out_desc(NCHUNK - 2, 0).wait()
    out_desc(NCHUNK - 1, 1).wait()


def kernel(attn_bias, spatial_pos, table):
    tabt = jnp.transpose(table).reshape(-1)
    mesh = plsc.VectorSubcoreMesh(core_axis_name="c", subcore_axis_name="s")
    f = pl.kernel(
        _sc_body,
        out_type=jax.ShapeDtypeStruct((B, H, N, N), jnp.float32),
        mesh=mesh,
        compiler_params=pltpu.CompilerParams(needs_layout_passes=False),
        scratch_types=[
            pltpu.VMEM((H * V,), jnp.float32),
            pltpu.VMEM((2, G, N), jnp.float32),
            pltpu.VMEM((2, G, N), jnp.int32),
            pltpu.VMEM((2, H, G, N), jnp.float32),
            pltpu.SemaphoreType.DMA,
            [pltpu.SemaphoreType.DMA, pltpu.SemaphoreType.DMA],
            [pltpu.SemaphoreType.DMA, pltpu.SemaphoreType.DMA],
            [pltpu.SemaphoreType.DMA, pltpu.SemaphoreType.DMA],
        ],
    )
    return f(attn_bias, spatial_pos.astype(jnp.int32), tabt)


# speedup vs baseline: 1.0891x; 1.0891x over previous
import jax
import jax.numpy as jnp
from jax import lax
from jax.experimental import pallas as pl
from jax.experimental.pallas import tpu as pltpu
from jax.experimental.pallas import tpu_sc as plsc

B = 8
H = 16
N = 512
V = 513
L = 16
R = 8
G = 2
ROWS_PER_TILE = (B * N) // 32
NCHUNK = ROWS_PER_TILE // G


def _sc_body(ab_hbm, sp_hbm, rep_hbm, out_hbm, tab_v, ab_v, idx_v, out_v,
             tab_sem, ab_sems, idx_sems, out_sems):
    nc = 2
    wid = lax.axis_index("s") * nc + lax.axis_index("c")
    b = wid // (N // ROWS_PER_TILE)
    i0 = (wid % (N // ROWS_PER_TILE)) * ROWS_PER_TILE

    pltpu.async_copy(rep_hbm, tab_v, tab_sem).wait()

    def in_descs(c, s):
        i = i0 + c * G
        return (
            pltpu.make_async_copy(ab_hbm.at[b, pl.ds(i, G), :], ab_v.at[s], ab_sems[s]),
            pltpu.make_async_copy(sp_hbm.at[b, pl.ds(i, G), :], idx_v.at[s], idx_sems[s]),
        )

    def out_desc(c, s):
        i = i0 + c * G
        return pltpu.make_async_copy(
            out_v.at[s], out_hbm.at[b, :, pl.ds(i, G), :], out_sems[s])

    def start_in(c, s):
        for d in in_descs(c, s):
            d.start()

    def wait_in(c, s):
        for d in in_descs(c, s):
            d.wait()

    lane8 = lax.iota(jnp.int32, L) & (R - 1)

    def compute(s):
        for r in range(G):
            @plsc.parallel_loop(0, N // L, unroll=4)
            def _vec_body(v):
                sl = pl.ds(v * L, L)
                idx8 = idx_v[s, r, sl] * R + lane8
                ab2 = ab_v[s, r, sl] * 2.0
                for h in range(H):
                    g = plsc.load_gather(tab_v, [idx8 + (h * V * R)])
                    out_v[s, h, r, sl] = ab2 + g

    start_in(0, 0)
    start_in(1, 1)

    def steady(c2, carry):
        for s in (0, 1):
            c = 2 * c2 + s

            @pl.when(c >= 2)
            def _wait_out():
                out_desc(c - 2, s).wait()

            wait_in(c, s)
            compute(s)
            out_desc(c, s).start()

            @pl.when(c < NCHUNK - 2)
            def _prefetch():
                start_in(c + 2, s)
        return carry

    lax.fori_loop(0, NCHUNK // 2, steady, 0)

    out_desc(NCHUNK - 2, 0).wait()
    out_desc(NCHUNK - 1, 1).wait()


def kernel(attn_bias, spatial_pos, table):
    rep = jnp.broadcast_to(jnp.transpose(table)[:, :, None], (H, V, R)).reshape(-1)
    mesh = plsc.VectorSubcoreMesh(core_axis_name="c", subcore_axis_name="s")
    f = pl.kernel(
        _sc_body,
        out_type=jax.ShapeDtypeStruct((B, H, N, N), jnp.float32),
        mesh=mesh,
        compiler_params=pltpu.CompilerParams(needs_layout_passes=False),
        scratch_types=[
            pltpu.VMEM((H * V * R,), jnp.float32),
            pltpu.VMEM((2, G, N), jnp.float32),
            pltpu.VMEM((2, G, N), jnp.int32),
            pltpu.VMEM((2, H, G, N), jnp.float32),
            pltpu.SemaphoreType.DMA,
            [pltpu.SemaphoreType.DMA, pltpu.SemaphoreType.DMA],
            [pltpu.SemaphoreType.DMA, pltpu.SemaphoreType.DMA],
            [pltpu.SemaphoreType.DMA, pltpu.SemaphoreType.DMA],
        ],
    )
    return f(attn_bias, spatial_pos.astype(jnp.int32), rep)


# speedup vs baseline: 1.6438x; 1.5094x over previous
import jax
import jax.numpy as jnp
from jax import lax
from jax.experimental import pallas as pl
from jax.experimental.pallas import tpu as pltpu
from jax.experimental.pallas import tpu_sc as plsc

B = 8
H = 16
N = 512
V = 513
VP = 520
L = 16
R = 8
G = 2
ROWS_PER_TILE = (B * N) // 32
NCHUNK = ROWS_PER_TILE // G


def _sc_body(ab_hbm, sp_hbm, rep_hbm, out_hbm, tab_v, ab_v, idx_v, out_v,
             tab_sem, ab_sems, idx_sems, out_sems):
    nc = 2
    wid = lax.axis_index("s") * nc + lax.axis_index("c")
    b = wid // (N // ROWS_PER_TILE)
    i0 = (wid % (N // ROWS_PER_TILE)) * ROWS_PER_TILE

    pltpu.async_copy(rep_hbm, tab_v, tab_sem).wait()

    def in_descs(c, s):
        i = i0 + c * G
        return (
            pltpu.make_async_copy(ab_hbm.at[b, pl.ds(i, G), :], ab_v.at[s], ab_sems[s]),
            pltpu.make_async_copy(sp_hbm.at[b, pl.ds(i, G), :], idx_v.at[s], idx_sems[s]),
        )

    def out_desc(c, s):
        i = i0 + c * G
        return pltpu.make_async_copy(
            out_v.at[s], out_hbm.at[b, :, pl.ds(i, G), :], out_sems[s])

    def start_in(c, s):
        for d in in_descs(c, s):
            d.start()

    def wait_in(c, s):
        for d in in_descs(c, s):
            d.wait()

    def compute(s):
        for r in range(G):
            @plsc.parallel_loop(0, N // L, unroll=4)
            def _vec_body(v):
                sl = pl.ds(v * L, L)
                idx = idx_v[s, r, sl]
                ab2 = ab_v[s, r, sl] * 2.0
                for h in range(H):
                    g = plsc.load_gather(tab_v.at[pl.ds(h * VP, VP)], [idx])
                    out_v[s, h, r, sl] = ab2 + g

    start_in(0, 0)
    start_in(1, 1)

    def steady(c2, carry):
        for s in (0, 1):
            c = 2 * c2 + s

            @pl.when(c >= 2)
            def _wait_out():
                out_desc(c - 2, s).wait()

            wait_in(c, s)
            compute(s)
            out_desc(c, s).start()

            @pl.when(c < NCHUNK - 2)
            def _prefetch():
                start_in(c + 2, s)
        return carry

    lax.fori_loop(0, NCHUNK // 2, steady, 0)

    out_desc(NCHUNK - 2, 0).wait()
    out_desc(NCHUNK - 1, 1).wait()


def kernel(attn_bias, spatial_pos, table):
    rep = jnp.pad(jnp.transpose(table), ((0, 0), (0, VP - V))).reshape(-1)
    mesh = plsc.VectorSubcoreMesh(core_axis_name="c", subcore_axis_name="s")
    f = pl.kernel(
        _sc_body,
        out_type=jax.ShapeDtypeStruct((B, H, N, N), jnp.float32),
        mesh=mesh,
        compiler_params=pltpu.CompilerParams(needs_layout_passes=False),
        scratch_types=[
            pltpu.VMEM((H * VP,), jnp.float32),
            pltpu.VMEM((2, G, N), jnp.float32),
            pltpu.VMEM((2, G, N), jnp.int32),
            pltpu.VMEM((2, H, G, N), jnp.float32),
            pltpu.SemaphoreType.DMA,
            [pltpu.SemaphoreType.DMA, pltpu.SemaphoreType.DMA],
            [pltpu.SemaphoreType.DMA, pltpu.SemaphoreType.DMA],
            [pltpu.SemaphoreType.DMA, pltpu.SemaphoreType.DMA],
        ],
    )
    return f(attn_bias, spatial_pos.astype(jnp.int32), rep)
